# baseline (speedup 1.0000x reference)
import jax
import jax.numpy as jnp
from jax.experimental import pallas as pl
from jax.experimental.pallas import tpu as pltpu

_CHUNK = 256


def _body(x_ref, mu_ref, logD_ref, eps_ref, theta_ref,
          out_ref, loss_ref, tn8_ref, tn32_ref, acc_ref):
    b = pl.program_id(0)
    nb = pl.num_programs(0)
    S = x_ref.shape[1]

    std = jnp.exp(0.5 * logD_ref[...])
    z_b = mu_ref[...] + eps_ref[0] * std

    @pl.when(b == 0)
    def _init():
        th = theta_ref[...]
        norm = jnp.sqrt(jnp.sum(th * th, axis=1, keepdims=True))
        tn = th / norm
        tn32_ref[...] = tn
        tn8_ref[...] = tn.astype(jnp.float8_e4m3fn)
        acc_ref[...] = jnp.zeros_like(acc_ref)

    tn8 = tn8_ref[...]
    zp = jax.lax.dot_general(
        z_b, tn32_ref[...],
        (((1,), (1,)), ((), ())), preferred_element_type=jnp.float32
    )

    col = jnp.zeros_like(acc_ref)
    for c in range(S // _CHUNK):
        sl = slice(c * _CHUNK, (c + 1) * _CHUNK)
        x_c = x_ref[0, sl, :]
        out_ref[0, sl, :] = x_c * z_b
        proj = jax.lax.dot_general(
            x_c.astype(jnp.float8_e4m3fn), tn8,
            (((1,), (1,)), ((), ())), preferred_element_type=jnp.float32
        )
        d = proj - zp
        col = col + jnp.sum(d * d, axis=0, keepdims=True)

    acc_ref[...] += col

    @pl.when(b == nb - 1)
    def _fin():
        loss_ref[...] = jnp.sum(acc_ref[...], axis=1, keepdims=True)


def kernel(x, post_z_mu, post_z_logD, eps, theta_raw):
    B, S, D = x.shape
    P = theta_raw.shape[0]
    mu2 = post_z_mu.reshape(1, D)
    logD2 = post_z_logD.reshape(1, D)
    eps3 = eps.reshape(B, 1, D)

    out, loss = pl.pallas_call(
        _body,
        grid=(B,),
        in_specs=[
            pl.BlockSpec((1, S, D), lambda b: (b, 0, 0)),
            pl.BlockSpec((1, D), lambda b: (0, 0)),
            pl.BlockSpec((1, D), lambda b: (0, 0)),
            pl.BlockSpec((1, 1, D), lambda b: (b, 0, 0)),
            pl.BlockSpec((P, D), lambda b: (0, 0)),
        ],
        out_specs=[
            pl.BlockSpec((1, S, D), lambda b: (b, 0, 0)),
            pl.BlockSpec((1, 1), lambda b: (0, 0)),
        ],
        out_shape=[
            jax.ShapeDtypeStruct((B, S, D), jnp.float32),
            jax.ShapeDtypeStruct((1, 1), jnp.float32),
        ],
        scratch_shapes=[
            pltpu.VMEM((P, D), jnp.float8_e4m3fn),
            pltpu.VMEM((P, D), jnp.float32),
            pltpu.VMEM((1, P), jnp.float32),
        ],
    )(x, mu2, logD2, eps3, theta_raw)

    w_loss = loss[0, 0] * (1.0 / (B * S * P))
    return out, w_loss


# speedup vs baseline: 1.0070x; 1.0070x over previous
import jax
import jax.numpy as jnp
from jax.experimental import pallas as pl
from jax.experimental.pallas import tpu as pltpu

_CHUNK = 1024


def _body(x_ref, mu_ref, logD_ref, eps_ref, theta_ref,
          out_ref, loss_ref, tn8_ref, tn32_ref, acc_ref):
    b = pl.program_id(0)
    nb = pl.num_programs(0)
    S = x_ref.shape[1]

    std = jnp.exp(0.5 * logD_ref[...])
    z_b = mu_ref[...] + eps_ref[0] * std

    @pl.when(b == 0)
    def _init():
        th = theta_ref[...]
        norm = jnp.sqrt(jnp.sum(th * th, axis=1, keepdims=True))
        tn = th / norm
        tn32_ref[...] = tn
        tn8_ref[...] = tn.astype(jnp.float8_e4m3fn)
        acc_ref[...] = jnp.zeros_like(acc_ref)

    tn8 = tn8_ref[...]
    zp = jax.lax.dot_general(
        z_b, tn32_ref[...],
        (((1,), (1,)), ((), ())), preferred_element_type=jnp.float32
    )

    col = jnp.zeros_like(acc_ref)
    for c in range(S // _CHUNK):
        sl = slice(c * _CHUNK, (c + 1) * _CHUNK)
        x_c = x_ref[0, sl, :]
        out_ref[0, sl, :] = x_c * z_b
        proj = jax.lax.dot_general(
            x_c.astype(jnp.float8_e4m3fn), tn8,
            (((1,), (1,)), ((), ())), preferred_element_type=jnp.float32
        )
        d = proj - zp
        col = col + jnp.sum(d * d, axis=0, keepdims=True)

    acc_ref[...] += col

    @pl.when(b == nb - 1)
    def _fin():
        loss_ref[...] = jnp.sum(acc_ref[...], axis=1, keepdims=True)


def kernel(x, post_z_mu, post_z_logD, eps, theta_raw):
    B, S, D = x.shape
    P = theta_raw.shape[0]
    mu2 = post_z_mu.reshape(1, D)
    logD2 = post_z_logD.reshape(1, D)
    eps3 = eps.reshape(B, 1, D)

    out, loss = pl.pallas_call(
        _body,
        grid=(B,),
        in_specs=[
            pl.BlockSpec((1, S, D), lambda b: (b, 0, 0)),
            pl.BlockSpec((1, D), lambda b: (0, 0)),
            pl.BlockSpec((1, D), lambda b: (0, 0)),
            pl.BlockSpec((1, 1, D), lambda b: (b, 0, 0)),
            pl.BlockSpec((P, D), lambda b: (0, 0)),
        ],
        out_specs=[
            pl.BlockSpec((1, S, D), lambda b: (b, 0, 0)),
            pl.BlockSpec((1, 1), lambda b: (0, 0)),
        ],
        out_shape=[
            jax.ShapeDtypeStruct((B, S, D), jnp.float32),
            jax.ShapeDtypeStruct((1, 1), jnp.float32),
        ],
        scratch_shapes=[
            pltpu.VMEM((P, D), jnp.float8_e4m3fn),
            pltpu.VMEM((P, D), jnp.float32),
            pltpu.VMEM((1, P), jnp.float32),
        ],
    )(x, mu2, logD2, eps3, theta_raw)

    w_loss = loss[0, 0] * (1.0 / (B * S * P))
    return out, w_loss
